# baseline (speedup 1.0000x reference)
import functools

import jax
import jax.numpy as jnp
from jax import lax
from jax.experimental import pallas as pl
from jax.experimental.pallas import tpu as pltpu
from jax.experimental.pallas import tpu_sc as plsc

_EMBEDDING_DIM = 64
_BATCH = 16384
_MARGIN = 1.0

_NC = 2
_NS = 16
_NW = _NC * _NS
_BPW = _BATCH // _NW
_CHUNK = 128
_NCHUNK = _BPW // _CHUNK
_L = 16


def _tec_body(pos_hbm, neg_hbm, ent_hbm, rel_hbm, out_hbm,
              ih_v, ir_v, it_v, jh_v, jr_v, jt_v,
              ph_v, pr_v, pt_v, nh_v, nr_v, nt_v,
              acc_v, sem):
    wid = lax.axis_index("s") * _NC + lax.axis_index("c")
    base = wid * _BPW
    lanes = lax.iota(jnp.int32, _L)
    zero = jnp.zeros((_L,), jnp.float32)
    loss = zero
    for g in range(_NCHUNK):
        off = base + g * _CHUNK
        pltpu.sync_copy(pos_hbm.at[pl.ds(off, _CHUNK)], ih_v)
        pltpu.sync_copy(pos_hbm.at[pl.ds(_BATCH + off, _CHUNK)], ir_v)
        pltpu.sync_copy(pos_hbm.at[pl.ds(2 * _BATCH + off, _CHUNK)], it_v)
        pltpu.sync_copy(neg_hbm.at[pl.ds(off, _CHUNK)], jh_v)
        pltpu.sync_copy(neg_hbm.at[pl.ds(_BATCH + off, _CHUNK)], jr_v)
        pltpu.sync_copy(neg_hbm.at[pl.ds(2 * _BATCH + off, _CHUNK)], jt_v)
        copies = [
            pltpu.async_copy(ent_hbm.at[ih_v], ph_v, sem),
            pltpu.async_copy(rel_hbm.at[ir_v], pr_v, sem),
            pltpu.async_copy(ent_hbm.at[it_v], pt_v, sem),
            pltpu.async_copy(ent_hbm.at[jh_v], nh_v, sem),
            pltpu.async_copy(rel_hbm.at[jr_v], nr_v, sem),
            pltpu.async_copy(ent_hbm.at[jt_v], nt_v, sem),
        ]
        for cp in copies:
            cp.wait()

        def group_body(j0, loss_c):
            rows = j0 * _L + lanes

            def d_body(d, carry):
                pd, nd = carry
                cols = jnp.full((_L,), d, jnp.int32)
                hp = plsc.load_gather(ph_v, [rows, cols])
                rp = plsc.load_gather(pr_v, [rows, cols])
                tp = plsc.load_gather(pt_v, [rows, cols])
                hn = plsc.load_gather(nh_v, [rows, cols])
                rn = plsc.load_gather(nr_v, [rows, cols])
                tn = plsc.load_gather(nt_v, [rows, cols])
                pd = pd + jnp.abs(hp + rp - tp)
                nd = nd + jnp.abs(hn + rn - tn)
                return pd, nd

            pd, nd = lax.fori_loop(0, _EMBEDDING_DIM, d_body, (zero, zero))
            return loss_c + jnp.maximum(pd - nd + _MARGIN, 0.0)

        loss = lax.fori_loop(0, _CHUNK // _L, group_body, loss)
    acc_v[...] = loss * (1.0 / _BATCH)
    pltpu.sync_copy(acc_v, out_hbm.at[wid])


@jax.jit
def kernel(positive_triples, negative_triples, entity_weight, relation_weight):
    pos = positive_triples.reshape(-1)
    neg = negative_triples.reshape(-1)
    mesh = plsc.VectorSubcoreMesh(core_axis_name="c", subcore_axis_name="s")
    f = functools.partial(
        pl.kernel,
        mesh=mesh,
        compiler_params=pltpu.CompilerParams(
            needs_layout_passes=False, use_tc_tiling_on_sc=False
        ),
        out_type=jax.ShapeDtypeStruct((_NW, _L), jnp.float32),
        scratch_types=(
            [pltpu.VMEM((_CHUNK,), jnp.int32)] * 6
            + [pltpu.VMEM((_CHUNK, _EMBEDDING_DIM), jnp.float32)] * 6
            + [pltpu.VMEM((_L,), jnp.float32), pltpu.SemaphoreType.DMA]
        ),
    )(_tec_body)
    partial = f(pos, neg, entity_weight, relation_weight)
    return jnp.sum(partial)


# speedup vs baseline: 1.0762x; 1.0762x over previous
import functools

import jax
import jax.numpy as jnp
from jax import lax
from jax.experimental import pallas as pl
from jax.experimental.pallas import tpu as pltpu
from jax.experimental.pallas import tpu_sc as plsc

_EMBEDDING_DIM = 64
_BATCH = 16384
_MARGIN = 1.0

_NC = 2
_NS = 16
_NW = _NC * _NS
_BPW = _BATCH // _NW
_CHUNK = 128
_NCHUNK = _BPW // _CHUNK
_L = 16
_UNROLL = 4


def _tec_body(pos_hbm, neg_hbm, ent_hbm, rel_hbm, out_hbm,
              ih_v, ir_v, it_v, jh_v, jr_v, jt_v,
              ph0, pr0, pt0, nh0, nr0, nt0,
              ph1, pr1, pt1, nh1, nr1, nt1,
              acc_v, sem0, sem1):
    wid = lax.axis_index("s") * _NC + lax.axis_index("c")
    base = wid * _BPW
    lanes = lax.iota(jnp.int32, _L)
    zero = jnp.zeros((_L,), jnp.float32)

    pltpu.sync_copy(pos_hbm.at[pl.ds(base, _BPW)], ih_v)
    pltpu.sync_copy(pos_hbm.at[pl.ds(_BATCH + base, _BPW)], ir_v)
    pltpu.sync_copy(pos_hbm.at[pl.ds(2 * _BATCH + base, _BPW)], it_v)
    pltpu.sync_copy(neg_hbm.at[pl.ds(base, _BPW)], jh_v)
    pltpu.sync_copy(neg_hbm.at[pl.ds(_BATCH + base, _BPW)], jr_v)
    pltpu.sync_copy(neg_hbm.at[pl.ds(2 * _BATCH + base, _BPW)], jt_v)

    bufsets = ((ph0, pr0, pt0, nh0, nr0, nt0), (ph1, pr1, pt1, nh1, nr1, nt1))
    sems = (sem0, sem1)

    def issue(g, bufs, sem):
        s = pl.ds(g * _CHUNK, _CHUNK)
        return [
            pltpu.async_copy(ent_hbm.at[ih_v.at[s]], bufs[0], sem),
            pltpu.async_copy(rel_hbm.at[ir_v.at[s]], bufs[1], sem),
            pltpu.async_copy(ent_hbm.at[it_v.at[s]], bufs[2], sem),
            pltpu.async_copy(ent_hbm.at[jh_v.at[s]], bufs[3], sem),
            pltpu.async_copy(rel_hbm.at[jr_v.at[s]], bufs[4], sem),
            pltpu.async_copy(ent_hbm.at[jt_v.at[s]], bufs[5], sem),
        ]

    def compute_chunk(bufs, loss_in):
        ph, pr, pt, nh, nr, nt = bufs

        def group(j0, loss_c):
            rows = j0 * _L + lanes

            def dstep(i, carry):
                accs = list(carry)
                d0 = i * _UNROLL
                for k in range(_UNROLL):
                    cols = jnp.full((_L,), d0 + k, jnp.int32)
                    hp = plsc.load_gather(ph, [rows, cols])
                    rp = plsc.load_gather(pr, [rows, cols])
                    tp = plsc.load_gather(pt, [rows, cols])
                    hn = plsc.load_gather(nh, [rows, cols])
                    rn = plsc.load_gather(nr, [rows, cols])
                    tn = plsc.load_gather(nt, [rows, cols])
                    accs[k] = accs[k] + jnp.abs(hp + rp - tp)
                    accs[_UNROLL + k] = accs[_UNROLL + k] + jnp.abs(hn + rn - tn)
                return tuple(accs)

            accs = lax.fori_loop(
                0, _EMBEDDING_DIM // _UNROLL, dstep, (zero,) * (2 * _UNROLL)
            )
            pd = (accs[0] + accs[1]) + (accs[2] + accs[3])
            nd = (accs[4] + accs[5]) + (accs[6] + accs[7])
            return loss_c + jnp.maximum(pd - nd + _MARGIN, 0.0)

        return lax.fori_loop(0, _CHUNK // _L, group, loss_in)

    loss = zero
    pend = issue(0, bufsets[0], sems[0])
    for g in range(_NCHUNK):
        for cp in pend:
            cp.wait()
        cur = bufsets[g % 2]
        if g + 1 < _NCHUNK:
            pend = issue(g + 1, bufsets[(g + 1) % 2], sems[(g + 1) % 2])
        loss = compute_chunk(cur, loss)

    acc_v[...] = loss * (1.0 / _BATCH)
    pltpu.sync_copy(acc_v, out_hbm.at[wid])


@jax.jit
def kernel(positive_triples, negative_triples, entity_weight, relation_weight):
    pos = positive_triples.reshape(-1)
    neg = negative_triples.reshape(-1)
    mesh = plsc.VectorSubcoreMesh(core_axis_name="c", subcore_axis_name="s")
    f = functools.partial(
        pl.kernel,
        mesh=mesh,
        compiler_params=pltpu.CompilerParams(
            needs_layout_passes=False, use_tc_tiling_on_sc=False
        ),
        out_type=jax.ShapeDtypeStruct((_NW, _L), jnp.float32),
        scratch_types=(
            [pltpu.VMEM((_BPW,), jnp.int32)] * 6
            + [pltpu.VMEM((_CHUNK, _EMBEDDING_DIM), jnp.float32)] * 12
            + [pltpu.VMEM((_L,), jnp.float32),
               pltpu.SemaphoreType.DMA, pltpu.SemaphoreType.DMA]
        ),
    )(_tec_body)
    partial = f(pos, neg, entity_weight, relation_weight)
    return jnp.sum(partial)


# speedup vs baseline: 1.7475x; 1.6238x over previous
import functools

import jax
import jax.numpy as jnp
from jax import lax
from jax.experimental import pallas as pl
from jax.experimental.pallas import tpu as pltpu
from jax.experimental.pallas import tpu_sc as plsc

_EMBEDDING_DIM = 64
_BATCH = 16384
_MARGIN = 1.0

_NC = 2
_NS = 16
_NW = _NC * _NS
_BPW = _BATCH // _NW
_CHUNK = 128
_NCHUNK = _BPW // _CHUNK
_L = 16
_UNROLL = 4


def _tec_body(pos_hbm, neg_hbm, ent_hbm, rel_hbm, out_hbm,
              ih_v, ir_v, it_v, jh_v, jr_v, jt_v,
              ph0, pr0, pt0, nh0, nr0, nt0,
              ph1, pr1, pt1, nh1, nr1, nt1,
              acc_v, sem0, sem1):
    wid = lax.axis_index("s") * _NC + lax.axis_index("c")
    base = wid * _BPW
    lanes = lax.iota(jnp.int32, _L)
    zero = jnp.zeros((_L,), jnp.float32)

    pltpu.sync_copy(pos_hbm.at[pl.ds(base, _BPW)], ih_v)
    pltpu.sync_copy(pos_hbm.at[pl.ds(_BATCH + base, _BPW)], ir_v)
    pltpu.sync_copy(pos_hbm.at[pl.ds(2 * _BATCH + base, _BPW)], it_v)
    pltpu.sync_copy(neg_hbm.at[pl.ds(base, _BPW)], jh_v)
    pltpu.sync_copy(neg_hbm.at[pl.ds(_BATCH + base, _BPW)], jr_v)
    pltpu.sync_copy(neg_hbm.at[pl.ds(2 * _BATCH + base, _BPW)], jt_v)

    bufsets = ((ph0, pr0, pt0, nh0, nr0, nt0), (ph1, pr1, pt1, nh1, nr1, nt1))
    sems = (sem0, sem1)

    def issue(g, bufs, sem):
        s = pl.ds(g * _CHUNK, _CHUNK)
        return [
            pltpu.async_copy(ent_hbm.at[ih_v.at[s]], bufs[0], sem),
            pltpu.async_copy(rel_hbm.at[ir_v.at[s]], bufs[1], sem),
            pltpu.async_copy(ent_hbm.at[it_v.at[s]], bufs[2], sem),
            pltpu.async_copy(ent_hbm.at[jh_v.at[s]], bufs[3], sem),
            pltpu.async_copy(rel_hbm.at[jr_v.at[s]], bufs[4], sem),
            pltpu.async_copy(ent_hbm.at[jt_v.at[s]], bufs[5], sem),
        ]

    def compute_chunk(bufs, loss_in):
        ph, pr, pt, nh, nr, nt = bufs

        def group(j0, loss_c):
            rows = j0 * _L + lanes

            def dstep(i, carry):
                accs = list(carry)
                d0 = i * _UNROLL
                for k in range(_UNROLL):
                    cols = jnp.bitwise_and(
                        d0 + k + lanes, _EMBEDDING_DIM - 1
                    )
                    hp = plsc.load_gather(ph, [rows, cols])
                    rp = plsc.load_gather(pr, [rows, cols])
                    tp = plsc.load_gather(pt, [rows, cols])
                    hn = plsc.load_gather(nh, [rows, cols])
                    rn = plsc.load_gather(nr, [rows, cols])
                    tn = plsc.load_gather(nt, [rows, cols])
                    accs[k] = accs[k] + jnp.abs(hp + rp - tp)
                    accs[_UNROLL + k] = accs[_UNROLL + k] + jnp.abs(hn + rn - tn)
                return tuple(accs)

            accs = lax.fori_loop(
                0, _EMBEDDING_DIM // _UNROLL, dstep, (zero,) * (2 * _UNROLL)
            )
            pd = (accs[0] + accs[1]) + (accs[2] + accs[3])
            nd = (accs[4] + accs[5]) + (accs[6] + accs[7])
            return loss_c + jnp.maximum(pd - nd + _MARGIN, 0.0)

        return lax.fori_loop(0, _CHUNK // _L, group, loss_in)

    loss = zero
    pend = issue(0, bufsets[0], sems[0])
    for g in range(_NCHUNK):
        for cp in pend:
            cp.wait()
        cur = bufsets[g % 2]
        if g + 1 < _NCHUNK:
            pend = issue(g + 1, bufsets[(g + 1) % 2], sems[(g + 1) % 2])
        loss = compute_chunk(cur, loss)

    acc_v[...] = loss * (1.0 / _BATCH)
    pltpu.sync_copy(acc_v, out_hbm.at[wid])


@jax.jit
def kernel(positive_triples, negative_triples, entity_weight, relation_weight):
    pos = positive_triples.reshape(-1)
    neg = negative_triples.reshape(-1)
    mesh = plsc.VectorSubcoreMesh(core_axis_name="c", subcore_axis_name="s")
    f = functools.partial(
        pl.kernel,
        mesh=mesh,
        compiler_params=pltpu.CompilerParams(
            needs_layout_passes=False, use_tc_tiling_on_sc=False
        ),
        out_type=jax.ShapeDtypeStruct((_NW, _L), jnp.float32),
        scratch_types=(
            [pltpu.VMEM((_BPW,), jnp.int32)] * 6
            + [pltpu.VMEM((_CHUNK, _EMBEDDING_DIM), jnp.float32)] * 12
            + [pltpu.VMEM((_L,), jnp.float32),
               pltpu.SemaphoreType.DMA, pltpu.SemaphoreType.DMA]
        ),
    )(_tec_body)
    partial = f(pos, neg, entity_weight, relation_weight)
    return jnp.sum(partial)
